# Initial kernel scaffold; baseline (speedup 1.0000x reference)
import jax
import jax.numpy as jnp
from jax.experimental import pallas as pl


def kernel(h, e, edge_index, W_h, W_e, W_p, b_p, W_a, gamma_h, beta_h, gamma_e, beta_e):
    raise NotImplementedError("write your pallas kernel here")


# speedup vs baseline: 3.0384x; 3.0384x over previous
import functools

import jax
import jax.numpy as jnp
from jax import lax
from jax.experimental import pallas as pl
from jax.experimental.pallas import tpu as pltpu
from jax.experimental.pallas import tpu_sc as plsc

N = 10000
E = 320000
D = 128

NC = 2
NS = 16
NW = NC * NS
EPW = E // NW
CH = 80
NCHUNK = EPW // CH
NPAD = 10240
RPS = NPAD // NS

_mesh = functools.partial(plsc.VectorSubcoreMesh,
                          core_axis_name="c", subcore_axis_name="s")


def _gather_body(zh_hbm, src_hbm, dst_hbm, zs_out, zd_out, idx_v, rows_v, sem):
    wid = lax.axis_index("s") * NC + lax.axis_index("c")
    base = wid * EPW

    def body(j, carry):
        off = base + j * CH
        pltpu.sync_copy(src_hbm.at[pl.ds(off, CH)], idx_v)
        pltpu.async_copy(zh_hbm.at[idx_v], rows_v, sem).wait()
        pltpu.sync_copy(rows_v, zs_out.at[pl.ds(off, CH)])
        pltpu.sync_copy(dst_hbm.at[pl.ds(off, CH)], idx_v)
        pltpu.async_copy(zh_hbm.at[idx_v], rows_v, sem).wait()
        pltpu.sync_copy(rows_v, zd_out.at[pl.ds(off, CH)])
        return carry

    lax.fori_loop(0, NCHUNK, body, 0)


def _sc_gather(z_h, src, dst):
    f = pl.kernel(
        _gather_body,
        mesh=_mesh(),
        out_type=[jax.ShapeDtypeStruct((E, D), jnp.float32),
                  jax.ShapeDtypeStruct((E, D), jnp.float32)],
        scratch_types=[pltpu.VMEM((CH,), jnp.int32),
                       pltpu.VMEM((CH, D), jnp.float32),
                       pltpu.SemaphoreType.DMA],
    )
    return f(z_h, src, dst)


def _scatter_body(w_hbm, exb_hbm, dsti_hbm, z128_hbm, z16_hbm, s_out, d_out,
                  idx_v, w_v, e_v, s_sh, d_sh):
    c = lax.axis_index("c")
    s = lax.axis_index("s")
    wid = s * NC + c
    zbase = s * RPS
    pltpu.sync_copy(z128_hbm, s_sh.at[pl.ds(zbase, RPS)])
    pltpu.sync_copy(z16_hbm, d_sh.at[pl.ds(zbase, RPS)])
    base = wid * EPW

    def body(j, carry):
        off = base + j * CH
        pltpu.sync_copy(dsti_hbm.at[pl.ds(off, CH)], idx_v)
        pltpu.sync_copy(w_hbm.at[pl.ds(off, CH)], w_v)
        pltpu.sync_copy(exb_hbm.at[pl.ds(off, CH)], e_v)
        return carry

    lax.fori_loop(0, NCHUNK, body, 0)
    obase = c * NPAD + zbase
    pltpu.sync_copy(s_sh.at[pl.ds(zbase, RPS)], s_out.at[pl.ds(obase, RPS)])
    pltpu.sync_copy(d_sh.at[pl.ds(zbase, RPS)], d_out.at[pl.ds(obase, RPS)])


def _sc_scatter(w, exb, dst):
    z128 = jnp.zeros((RPS, D), jnp.float32)
    z16 = jnp.zeros((RPS, 16), jnp.float32)
    f = pl.kernel(
        _scatter_body,
        mesh=_mesh(),
        out_type=[jax.ShapeDtypeStruct((NC * NPAD, D), jnp.float32),
                  jax.ShapeDtypeStruct((NC * NPAD, 16), jnp.float32)],
        scratch_types=[pltpu.VMEM((CH,), jnp.int32),
                       pltpu.VMEM((CH, D), jnp.float32),
                       pltpu.VMEM((CH, 16), jnp.float32),
                       pltpu.VMEM_SHARED((NPAD, D), jnp.float32),
                       pltpu.VMEM_SHARED((NPAD, 16), jnp.float32)],
    )
    s_flat, d_flat = f(w, exb, dst, z128, z16)
    return (s_flat.reshape(NC, NPAD, D), d_flat.reshape(NC, NPAD, 16))


def _mm_body(h_ref, w_ref, o_ref):
    o_ref[...] = jnp.dot(h_ref[...], w_ref[...],
                         preferred_element_type=jnp.float32)


ET = 512
EGRID = E // ET


def _edge_body(e_ref, zs_ref, zd_ref, we_ref, ws_ref, wd_ref, bp_ref,
               ep_ref, w_ref, exb_ref, sum_ref, sumsq_ref):
    big = (jnp.dot(e_ref[...], we_ref[...], preferred_element_type=jnp.float32)
           + jnp.dot(zs_ref[...], ws_ref[...], preferred_element_type=jnp.float32)
           + jnp.dot(zd_ref[...], wd_ref[...], preferred_element_type=jnp.float32))
    ep = big[:, :D] + bp_ref[...]
    ep_ref[...] = ep
    exb = jnp.exp(jnp.maximum(big[:, D:D + 16], 0.0))
    exb_ref[...] = exb
    w_ref[...] = exb[:, 0:1] * zs_ref[...]
    ps = jnp.sum(ep, axis=0, keepdims=True)
    pq = jnp.sum(ep * ep, axis=0, keepdims=True)
    i = pl.program_id(0)

    @pl.when(i == 0)
    def _():
        sum_ref[...] = ps
        sumsq_ref[...] = pq

    @pl.when(i > 0)
    def _():
        sum_ref[...] += ps
        sumsq_ref[...] += pq


def _hfin_body(s_ref, d_ref, g_ref, b_ref, o_ref):
    stot = s_ref[...]
    den = d_ref[:, 0:1]
    hn = stot[:N] / jnp.maximum(den[:N], 0.5)
    mu = jnp.mean(hn, axis=0, keepdims=True)
    var = jnp.mean((hn - mu) * (hn - mu), axis=0, keepdims=True)
    inv = lax.rsqrt(var + 1e-5)
    o_ref[...] = jnp.maximum(g_ref[...] * (hn - mu) * inv + b_ref[...], 0.0)


def _efin_body(ep_ref, sum_ref, sumsq_ref, g_ref, b_ref, o_ref):
    mu = sum_ref[...] / E
    var = sumsq_ref[...] / E - mu * mu
    inv = lax.rsqrt(var + 1e-5)
    o_ref[...] = jnp.maximum(g_ref[...] * (ep_ref[...] - mu) * inv + b_ref[...],
                             0.0)


@jax.jit
def kernel(h, e, edge_index, W_h, W_e, W_p, b_p, W_a,
           gamma_h, beta_h, gamma_e, beta_e):
    src = edge_index[0]
    dst = edge_index[1]

    def pack(wp, wa):
        z = jnp.zeros((D, 2 * D), jnp.float32)
        z = z.at[:, :D].set(wp)
        z = z.at[:, D:D + 16].set(jnp.broadcast_to(wa, (D, 16)))
        return z

    we_pack = pack(W_e @ W_p[:D], W_e @ W_a[:D])
    ws_pack = pack(W_p[D:2 * D], W_a[D:2 * D])
    wd_pack = pack(W_p[2 * D:], W_a[2 * D:])
    bp2 = b_p.reshape(1, D)

    z_h = pl.pallas_call(
        _mm_body,
        out_shape=jax.ShapeDtypeStruct((N, D), jnp.float32),
    )(h, W_h)

    zs, zd = _sc_gather(z_h, src, dst)

    e_proj, w, exb, psum, psumsq = pl.pallas_call(
        _edge_body,
        grid=(EGRID,),
        in_specs=[
            pl.BlockSpec((ET, D), lambda i: (i, 0)),
            pl.BlockSpec((ET, D), lambda i: (i, 0)),
            pl.BlockSpec((ET, D), lambda i: (i, 0)),
            pl.BlockSpec((D, 2 * D), lambda i: (0, 0)),
            pl.BlockSpec((D, 2 * D), lambda i: (0, 0)),
            pl.BlockSpec((D, 2 * D), lambda i: (0, 0)),
            pl.BlockSpec((1, D), lambda i: (0, 0)),
        ],
        out_specs=[
            pl.BlockSpec((ET, D), lambda i: (i, 0)),
            pl.BlockSpec((ET, D), lambda i: (i, 0)),
            pl.BlockSpec((ET, 16), lambda i: (i, 0)),
            pl.BlockSpec((1, D), lambda i: (0, 0)),
            pl.BlockSpec((1, D), lambda i: (0, 0)),
        ],
        out_shape=[
            jax.ShapeDtypeStruct((E, D), jnp.float32),
            jax.ShapeDtypeStruct((E, D), jnp.float32),
            jax.ShapeDtypeStruct((E, 16), jnp.float32),
            jax.ShapeDtypeStruct((1, D), jnp.float32),
            jax.ShapeDtypeStruct((1, D), jnp.float32),
        ],
    )(e, zs, zd, we_pack, ws_pack, wd_pack, bp2)

    s_part = jax.ops.segment_sum(w, dst, num_segments=NPAD)
    d_part = jax.ops.segment_sum(exb, dst, num_segments=NPAD)

    h_out = pl.pallas_call(
        _hfin_body,
        out_shape=jax.ShapeDtypeStruct((N, D), jnp.float32),
    )(s_part, d_part, gamma_h.reshape(1, D), beta_h.reshape(1, D))

    e_out = pl.pallas_call(
        _efin_body,
        grid=(EGRID,),
        in_specs=[
            pl.BlockSpec((ET, D), lambda i: (i, 0)),
            pl.BlockSpec((1, D), lambda i: (0, 0)),
            pl.BlockSpec((1, D), lambda i: (0, 0)),
            pl.BlockSpec((1, D), lambda i: (0, 0)),
            pl.BlockSpec((1, D), lambda i: (0, 0)),
        ],
        out_specs=pl.BlockSpec((ET, D), lambda i: (i, 0)),
        out_shape=jax.ShapeDtypeStruct((E, D), jnp.float32),
    )(e_proj, psum, psumsq, gamma_e.reshape(1, D), beta_e.reshape(1, D))

    return h_out, e_out
